# Initial kernel scaffold; baseline (speedup 1.0000x reference)
import jax
import jax.numpy as jnp
from jax.experimental import pallas as pl


def kernel(scores, boxes):
    raise NotImplementedError("write your pallas kernel here")


# speedup vs baseline: 23.4324x; 23.4324x over previous
import functools

import jax
import jax.numpy as jnp
from jax import lax
from jax.experimental import pallas as pl
from jax.experimental.pallas import tpu as pltpu
from jax.experimental.pallas import tpu_sc as plsc

IOU_T = 0.5
SCORE_T = 0.05
NEGV = -1.0e9
N = 20000
NPAD = 20480
NBATCH = 4
NCLS = 20
MAXP = 100
MAXT = 300
PP = 112
L = 16
NL1 = 1280
NL2 = 80
M2 = PP * NCLS
M2PAD = 2304
ML1 = 144

_mesh = plsc.VectorSubcoreMesh(core_axis_name="c", subcore_axis_name="s")


_GATHER_DNUMS = lax.GatherDimensionNumbers(
    offset_dims=(), collapsed_slice_dims=(0,), start_index_map=(0,))


def _shuf(v, perm):
    return lax.gather(v, perm[:, None], _GATHER_DNUMS, (1,),
                      mode=lax.GatherScatterMode.PROMISE_IN_BOUNDS)


def _make_lane_ops():
    iota = lax.iota(jnp.int32, L)
    perms = [iota ^ s for s in (8, 4, 2, 1)]

    def splat_max(v):
        for p in perms:
            v = jnp.maximum(v, _shuf(v, p))
        return v

    def splat_min(v):
        for p in perms:
            v = jnp.minimum(v, _shuf(v, p))
        return v

    def lane(v, i):
        return _shuf(v, jnp.full((L,), i, jnp.int32))

    return iota, splat_max, splat_min, lane


@functools.partial(
    pl.kernel,
    out_type=(
        jax.ShapeDtypeStruct((NBATCH * NCLS * PP,), jnp.float32),
        jax.ShapeDtypeStruct((NBATCH * NCLS * PP * 4,), jnp.float32),
    ),
    mesh=_mesh,
    scratch_types=[
        pltpu.VMEM((NPAD,), jnp.float32),
        pltpu.VMEM((N * 4,), jnp.float32),
        pltpu.VMEM((NL1,), jnp.float32),
        pltpu.VMEM((NL2,), jnp.float32),
        pltpu.VMEM((PP,), jnp.float32),
        pltpu.VMEM((PP,), jnp.float32),
        pltpu.VMEM((PP,), jnp.float32),
        pltpu.VMEM((PP,), jnp.float32),
        pltpu.VMEM((PP,), jnp.float32),
        pltpu.VMEM((PP,), jnp.float32),
        pltpu.VMEM((PP * 4,), jnp.float32),
    ],
)
def _nms_phase1(scores_hbm, boxes_hbm, sels_hbm, selb_hbm,
                sw, bx, l1, l2, ay1, ax1, ay2, ax2, aar, outs, outb):
    wid = lax.axis_index("s") * 2 + lax.axis_index("c")
    iota, splat_max, splat_min, lane = _make_lane_ops()
    negv = jnp.full((L,), NEGV, jnp.float32)
    twov = jnp.full((L,), 2.0, jnp.float32)
    zerov = jnp.zeros((L,), jnp.float32)
    bigv = jnp.full((L,), 1 << 20, jnp.int32)

    def run_task(t, b, load_boxes):
        pltpu.sync_copy(scores_hbm.at[pl.ds(t * N, N)], sw.at[pl.ds(0, N)])

        @pl.when(load_boxes)
        def _():
            pltpu.sync_copy(boxes_hbm.at[pl.ds(b * (N * 4), N * 4)], bx)

        for j in range(N // L, NPAD // L):
            sw[pl.ds(j * L, L)] = negv

        def build(v, carry):
            asm = negv
            for r in range(L):
                u = sw[pl.ds(v * 256 + r * L, L)]
                asm = jnp.where(iota == r, splat_max(u), asm)
            l1[pl.ds(v * L, L)] = asm
            return carry

        lax.fori_loop(0, NL1 // L, build, jnp.int32(0))

        def build2(q, carry):
            asm = negv
            for r in range(L):
                u = l1[pl.ds(q * 256 + r * L, L)]
                asm = jnp.where(iota == r, splat_max(u), asm)
            l2[pl.ds(q * L, L)] = asm
            return carry

        lax.fori_loop(0, NL2 // L, build2, jnp.int32(0))

        for j in range(PP // L):
            ay1[pl.ds(j * L, L)] = twov
            ax1[pl.ds(j * L, L)] = twov
            ay2[pl.ds(j * L, L)] = twov
            ax2[pl.ds(j * L, L)] = twov
            aar[pl.ds(j * L, L)] = zerov
            outs[pl.ds(j * L, L)] = negv
        for j in range(PP * 4 // L):
            outb[pl.ds(j * L, L)] = zerov

        def body(carry):
            cnt, _ = carry
            v0 = l2[pl.ds(0, L)]
            v1 = l2[pl.ds(16, L)]
            v2 = l2[pl.ds(32, L)]
            v3 = l2[pl.ds(48, L)]
            v4 = l2[pl.ds(64, L)]
            mv = splat_max(jnp.maximum(jnp.maximum(v0, v1),
                                       jnp.maximum(jnp.maximum(v2, v3), v4)))
            kw = jnp.where(v0 == mv, iota, bigv)
            kw = jnp.minimum(kw, jnp.where(v1 == mv, iota + 16, bigv))
            kw = jnp.minimum(kw, jnp.where(v2 == mv, iota + 32, bigv))
            kw = jnp.minimum(kw, jnp.where(v3 == mv, iota + 48, bigv))
            kw = jnp.minimum(kw, jnp.where(v4 == mv, iota + 64, bigv))
            k = splat_min(kw)[0]
            l1v = l1[pl.ds(k * L, L)]
            f1v = splat_min(jnp.where(l1v == mv, iota, bigv))
            f1 = f1v[0]
            c = k * L + f1
            l0 = sw[pl.ds(c * L, L)]
            f0v = splat_min(jnp.where(l0 == mv, iota, bigv))
            f0 = f0v[0]
            idx = c * L + f0
            m = mv[0]
            valid = m > SCORE_T
            i4 = idx * 4
            qx = i4 // L * L
            lx = i4 % L
            vbx = bx[pl.ds(qx, L)]
            b0 = lane(vbx, lx)
            b1 = lane(vbx, lx + 1)
            b2 = lane(vbx, lx + 2)
            b3 = lane(vbx, lx + 3)
            cy1 = jnp.minimum(b0, b2)
            cy2 = jnp.maximum(b0, b2)
            cx1 = jnp.minimum(b1, b3)
            cx2 = jnp.maximum(b1, b3)
            car = (cy2 - cy1) * (cx2 - cx1)
            sup = jnp.zeros((L,), jnp.int32)
            for j in range(PP // L):
                y1a = ay1[pl.ds(j * L, L)]
                x1a = ax1[pl.ds(j * L, L)]
                y2a = ay2[pl.ds(j * L, L)]
                x2a = ax2[pl.ds(j * L, L)]
                ara = aar[pl.ds(j * L, L)]
                ih = jnp.maximum(jnp.minimum(cy2, y2a) - jnp.maximum(cy1, y1a), 0.0)
                iw = jnp.maximum(jnp.minimum(cx2, x2a) - jnp.maximum(cx1, x1a), 0.0)
                inter = ih * iw
                union = car + ara - inter
                iou = inter / jnp.maximum(union, 1e-8)
                sup = jnp.where(iou > IOU_T, 1, sup)
            nsup = splat_max(sup)[0]
            accept = jnp.logical_and(valid, nsup == 0)

            @pl.when(valid)
            def _():
                l0n = jnp.where(iota == f0, negv, l0)
                sw[pl.ds(c * L, L)] = l0n
                l1n = jnp.where(iota == f1, splat_max(l0n), l1v)
                l1[pl.ds(k * L, L)] = l1n
                q2 = k // L * L
                r2 = k % L
                u2 = l2[pl.ds(q2, L)]
                l2[pl.ds(q2, L)] = jnp.where(iota == r2, splat_max(l1n), u2)

            @pl.when(accept)
            def _():
                qa = cnt // L * L
                mska = iota == (cnt % L)

                def put(ref, sv):
                    v = ref[pl.ds(qa, L)]
                    ref[pl.ds(qa, L)] = jnp.where(mska, sv, v)

                put(ay1, cy1)
                put(ax1, cx1)
                put(ay2, cy2)
                put(ax2, cx2)
                put(aar, car)
                put(outs, mv)
                c4 = cnt * 4
                qb = c4 // L * L
                lb = c4 % L
                vb = outb[pl.ds(qb, L)]
                vb = jnp.where(iota == lb, b0, vb)
                vb = jnp.where(iota == lb + 1, b1, vb)
                vb = jnp.where(iota == lb + 2, b2, vb)
                vb = jnp.where(iota == lb + 3, b3, vb)
                outb[pl.ds(qb, L)] = vb

            ncnt = cnt + jnp.where(accept, 1, 0).astype(jnp.int32)
            go = jnp.logical_and(valid, ncnt < MAXP)
            return (ncnt, jnp.where(go, 1, 0).astype(jnp.int32))

        def step(i, carry):
            return lax.cond(carry[1] > 0, body, lambda c: c, carry)

        state = (jnp.int32(0), jnp.int32(1))
        for rs in (192, 832, 3264, 16192):
            state = lax.cond(
                state[1] > 0,
                lambda s, n=rs: lax.fori_loop(0, n, step, s),
                lambda s: s,
                state,
            )
        pltpu.sync_copy(outs, sels_hbm.at[pl.ds(t * PP, PP)])
        pltpu.sync_copy(outb, selb_hbm.at[pl.ds(t * PP * 4, PP * 4)])

    def task_loop(r, bprev):
        t = jnp.where(wid < 16, wid * 3 + r, 48 + (wid - 16) * 2 + r)
        act = jnp.logical_or(wid < 16, r < 2)
        b = t // NCLS
        bnew = jnp.where(act, b, bprev)

        @pl.when(act)
        def _():
            run_task(t, b, b != bprev)

        return bnew

    lax.fori_loop(0, 3, task_loop, jnp.int32(-1))


@functools.partial(
    pl.kernel,
    out_type=(
        jax.ShapeDtypeStruct((NBATCH * 304,), jnp.float32),
        jax.ShapeDtypeStruct((NBATCH * 1216,), jnp.float32),
        jax.ShapeDtypeStruct((NBATCH * 304,), jnp.float32),
        jax.ShapeDtypeStruct((NBATCH * 16,), jnp.int32),
    ),
    mesh=_mesh,
    scratch_types=[
        pltpu.VMEM((M2PAD,), jnp.float32),
        pltpu.VMEM((ML1,), jnp.float32),
        pltpu.VMEM((M2 * 4,), jnp.float32),
        pltpu.VMEM((304,), jnp.float32),
        pltpu.VMEM((1216,), jnp.float32),
        pltpu.VMEM((304,), jnp.float32),
        pltpu.VMEM((16,), jnp.int32),
    ],
)
def _merge_phase2(sels_hbm, selb_hbm, os_hbm, ob_hbm, ocl_hbm, ocnt_hbm,
                  s2, l1b, bx2, vouts, voutb, voutc, vcnt):
    wid = lax.axis_index("s") * 2 + lax.axis_index("c")
    iota, splat_max, splat_min, lane = _make_lane_ops()
    negv = jnp.full((L,), NEGV, jnp.float32)
    zerov = jnp.zeros((L,), jnp.float32)
    bigv = jnp.full((L,), 1 << 20, jnp.int32)

    @pl.when(wid < NBATCH)
    def _():
        b = wid
        pltpu.sync_copy(sels_hbm.at[pl.ds(b * M2, M2)], s2.at[pl.ds(0, M2)])
        pltpu.sync_copy(selb_hbm.at[pl.ds(b * M2 * 4, M2 * 4)], bx2)
        for j in range(M2 // L, M2PAD // L):
            s2[pl.ds(j * L, L)] = negv

        def build(q, carry):
            asm = negv
            for r in range(L):
                u = s2[pl.ds(q * 256 + r * L, L)]
                asm = jnp.where(iota == r, splat_max(u), asm)
            l1b[pl.ds(q * L, L)] = asm
            return carry

        lax.fori_loop(0, ML1 // L, build, jnp.int32(0))
        for j in range(304 // L):
            vouts[pl.ds(j * L, L)] = zerov
            voutc[pl.ds(j * L, L)] = zerov
        for j in range(1216 // L):
            voutb[pl.ds(j * L, L)] = zerov

        def step(i, cnt):
            u = [l1b[pl.ds(16 * j, L)] for j in range(ML1 // L)]
            m01 = jnp.maximum(u[0], u[1])
            m23 = jnp.maximum(u[2], u[3])
            m45 = jnp.maximum(u[4], u[5])
            m67 = jnp.maximum(u[6], u[7])
            mv = splat_max(jnp.maximum(jnp.maximum(jnp.maximum(m01, m23),
                                                   jnp.maximum(m45, m67)), u[8]))
            kw = bigv
            for j in range(ML1 // L):
                kw = jnp.minimum(kw, jnp.where(u[j] == mv, iota + 16 * j, bigv))
            k = splat_min(kw)[0]
            l0 = s2[pl.ds(k * L, L)]
            f0v = splat_min(jnp.where(l0 == mv, iota, bigv))
            f0 = f0v[0]
            idx = k * L + f0
            m = mv[0]
            valid = m > NEGV * 0.5

            @pl.when(valid)
            def _():
                l0n = jnp.where(iota == f0, negv, l0)
                s2[pl.ds(k * L, L)] = l0n
                qk = k // L * L
                rk = k % L
                uk = l1b[pl.ds(qk, L)]
                l1b[pl.ds(qk, L)] = jnp.where(iota == rk, splat_max(l0n), uk)
                cls = idx // PP
                clsv = jnp.full((L,), cls, jnp.int32).astype(jnp.float32)
                i4 = idx * 4
                qx = i4 // L * L
                lx = i4 % L
                vbx = bx2[pl.ds(qx, L)]
                vbx = jnp.minimum(jnp.maximum(vbx, 0.0), 1.0)
                b0 = lane(vbx, lx)
                b1 = lane(vbx, lx + 1)
                b2 = lane(vbx, lx + 2)
                b3 = lane(vbx, lx + 3)
                qo = i // L * L
                msko = iota == (i % L)
                vo = vouts[pl.ds(qo, L)]
                vouts[pl.ds(qo, L)] = jnp.where(msko, mv, vo)
                vc = voutc[pl.ds(qo, L)]
                voutc[pl.ds(qo, L)] = jnp.where(msko, clsv, vc)
                q4 = i * 4
                qb = q4 // L * L
                lb = q4 % L
                vb = voutb[pl.ds(qb, L)]
                vb = jnp.where(iota == lb, b0, vb)
                vb = jnp.where(iota == lb + 1, b1, vb)
                vb = jnp.where(iota == lb + 2, b2, vb)
                vb = jnp.where(iota == lb + 3, b3, vb)
                voutb[pl.ds(qb, L)] = vb

            return cnt + jnp.where(valid, 1, 0).astype(jnp.int32)

        cnt = lax.fori_loop(0, MAXT, step, jnp.int32(0))
        vcnt[pl.ds(0, L)] = jnp.where(iota == 0, jnp.full((L,), cnt, jnp.int32),
                                      jnp.zeros((L,), jnp.int32))
        pltpu.sync_copy(vouts, os_hbm.at[pl.ds(b * 304, 304)])
        pltpu.sync_copy(voutb, ob_hbm.at[pl.ds(b * 1216, 1216)])
        pltpu.sync_copy(voutc, ocl_hbm.at[pl.ds(b * 304, 304)])
        pltpu.sync_copy(vcnt, ocnt_hbm.at[pl.ds(b * L, L)])


def kernel(scores, boxes):
    scores_t = jnp.transpose(scores, (0, 2, 1)).reshape(-1)
    boxes_f = boxes[:, :, 0, :].reshape(-1)
    sels, selb = _nms_phase1(scores_t, boxes_f)
    osf, obf, oclf, ocnt = _merge_phase2(sels, selb)
    out_scores = osf.reshape(NBATCH, 304)[:, :MAXT]
    out_boxes = obf.reshape(NBATCH, 304, 4)[:, :MAXT, :]
    out_classes = oclf.reshape(NBATCH, 304)[:, :MAXT]
    valid_counts = ocnt.reshape(NBATCH, 16)[:, 0]
    return (out_scores, out_boxes, out_classes, valid_counts)


# speedup vs baseline: 23.6442x; 1.0090x over previous
import functools

import jax
import jax.numpy as jnp
from jax import lax
from jax.experimental import pallas as pl
from jax.experimental.pallas import tpu as pltpu
from jax.experimental.pallas import tpu_sc as plsc

IOU_T = 0.5
SCORE_T = 0.05
NEGV = -1.0e9
N = 20000
NPAD = 20480
NBATCH = 4
NCLS = 20
MAXP = 100
MAXT = 300
PP = 112
L = 16
NL1 = 1280
NL2 = 80
M2 = PP * NCLS
M2PAD = 2304
ML1 = 144
TPC = NBATCH * NCLS // 2

_mesh = plsc.VectorSubcoreMesh(core_axis_name="c", subcore_axis_name="s")

_GATHER_DNUMS = lax.GatherDimensionNumbers(
    offset_dims=(), collapsed_slice_dims=(0,), start_index_map=(0,))


def _shuf(v, perm):
    return lax.gather(v, perm[:, None], _GATHER_DNUMS, (1,),
                      mode=lax.GatherScatterMode.PROMISE_IN_BOUNDS)


def _make_lane_ops():
    iota = lax.iota(jnp.int32, L)
    perms = [iota ^ s for s in (8, 4, 2, 1)]

    def splat_max(v):
        for p in perms:
            v = jnp.maximum(v, _shuf(v, p))
        return v

    def splat_min(v):
        for p in perms:
            v = jnp.minimum(v, _shuf(v, p))
        return v

    def lane(v, i):
        return _shuf(v, jnp.full((L,), i, jnp.int32))

    return iota, splat_max, splat_min, lane


@functools.partial(
    pl.kernel,
    out_type=(
        jax.ShapeDtypeStruct((NBATCH * 304,), jnp.float32),
        jax.ShapeDtypeStruct((NBATCH * 1216,), jnp.float32),
        jax.ShapeDtypeStruct((NBATCH * 304,), jnp.float32),
        jax.ShapeDtypeStruct((NBATCH * 16,), jnp.int32),
    ),
    mesh=_mesh,
    scratch_types=[
        pltpu.VMEM((NPAD,), jnp.float32),
        pltpu.VMEM((N * 4,), jnp.float32),
        pltpu.VMEM((NL1,), jnp.float32),
        pltpu.VMEM((NL2,), jnp.float32),
        pltpu.VMEM((PP,), jnp.float32),
        pltpu.VMEM((PP,), jnp.float32),
        pltpu.VMEM((PP,), jnp.float32),
        pltpu.VMEM((PP,), jnp.float32),
        pltpu.VMEM((PP,), jnp.float32),
        pltpu.VMEM((PP,), jnp.float32),
        pltpu.VMEM((PP * 4,), jnp.float32),
        pltpu.VMEM_SHARED((TPC * PP,), jnp.float32),
        pltpu.VMEM_SHARED((TPC * PP * 4,), jnp.float32),
        pltpu.VMEM((M2PAD,), jnp.float32),
        pltpu.VMEM((ML1,), jnp.float32),
        pltpu.VMEM((L,), jnp.float32),
        pltpu.VMEM((M2 * 4,), jnp.float32),
        pltpu.VMEM((304,), jnp.float32),
        pltpu.VMEM((1216,), jnp.float32),
        pltpu.VMEM((304,), jnp.float32),
        pltpu.VMEM((16,), jnp.int32),
    ],
)
def _nms_fused(scores_hbm, boxes_hbm, os_hbm, ob_hbm, ocl_hbm, ocnt_hbm,
               sw, bx, l1, l2, ay1, ax1, ay2, ax2, aar, outs, outb,
               sels_sh, selb_sh,
               s2, l1b, l2b, bx2, vouts, voutb, voutc, vcnt):
    sc = lax.axis_index("c")
    lid = lax.axis_index("s")
    iota, splat_max, splat_min, lane = _make_lane_ops()
    negv = jnp.full((L,), NEGV, jnp.float32)
    twov = jnp.full((L,), 2.0, jnp.float32)
    zerov = jnp.zeros((L,), jnp.float32)
    bigv = jnp.full((L,), 1 << 20, jnp.int32)

    def run_task(t, b, load_boxes):
        lt = t - sc * TPC
        pltpu.sync_copy(scores_hbm.at[pl.ds(t * N, N)], sw.at[pl.ds(0, N)])

        @pl.when(load_boxes)
        def _():
            pltpu.sync_copy(boxes_hbm.at[pl.ds(b * (N * 4), N * 4)], bx)

        for j in range(N // L, NPAD // L):
            sw[pl.ds(j * L, L)] = negv

        def build(v, carry):
            asm = negv
            for r in range(L):
                u = sw[pl.ds(v * 256 + r * L, L)]
                asm = jnp.where(iota == r, splat_max(u), asm)
            l1[pl.ds(v * L, L)] = asm
            return carry

        lax.fori_loop(0, NL1 // L, build, jnp.int32(0))

        def build2(q, carry):
            asm = negv
            for r in range(L):
                u = l1[pl.ds(q * 256 + r * L, L)]
                asm = jnp.where(iota == r, splat_max(u), asm)
            l2[pl.ds(q * L, L)] = asm
            return carry

        lax.fori_loop(0, NL2 // L, build2, jnp.int32(0))

        for j in range(PP // L):
            ay1[pl.ds(j * L, L)] = twov
            ax1[pl.ds(j * L, L)] = twov
            ay2[pl.ds(j * L, L)] = twov
            ax2[pl.ds(j * L, L)] = twov
            aar[pl.ds(j * L, L)] = zerov
            outs[pl.ds(j * L, L)] = negv
        for j in range(PP * 4 // L):
            outb[pl.ds(j * L, L)] = zerov

        def body(carry):
            cnt, _ = carry
            v0 = l2[pl.ds(0, L)]
            v1 = l2[pl.ds(16, L)]
            v2 = l2[pl.ds(32, L)]
            v3 = l2[pl.ds(48, L)]
            v4 = l2[pl.ds(64, L)]
            mv = splat_max(jnp.maximum(jnp.maximum(v0, v1),
                                       jnp.maximum(jnp.maximum(v2, v3), v4)))
            kw = jnp.where(v0 == mv, iota, bigv)
            kw = jnp.minimum(kw, jnp.where(v1 == mv, iota + 16, bigv))
            kw = jnp.minimum(kw, jnp.where(v2 == mv, iota + 32, bigv))
            kw = jnp.minimum(kw, jnp.where(v3 == mv, iota + 48, bigv))
            kw = jnp.minimum(kw, jnp.where(v4 == mv, iota + 64, bigv))
            k = splat_min(kw)[0]
            l1v = l1[pl.ds(k * L, L)]
            f1v = splat_min(jnp.where(l1v == mv, iota, bigv))
            f1 = f1v[0]
            c = k * L + f1
            l0 = sw[pl.ds(c * L, L)]
            f0v = splat_min(jnp.where(l0 == mv, iota, bigv))
            f0 = f0v[0]
            idx = c * L + f0
            m = mv[0]
            valid = m > SCORE_T
            i4 = idx * 4
            qx = i4 // L * L
            lx = i4 % L
            vbx = bx[pl.ds(qx, L)]
            b0 = lane(vbx, lx)
            b1 = lane(vbx, lx + 1)
            b2 = lane(vbx, lx + 2)
            b3 = lane(vbx, lx + 3)
            cy1 = jnp.minimum(b0, b2)
            cy2 = jnp.maximum(b0, b2)
            cx1 = jnp.minimum(b1, b3)
            cx2 = jnp.maximum(b1, b3)
            car = (cy2 - cy1) * (cx2 - cx1)
            sup = jnp.zeros((L,), jnp.int32)
            for j in range(PP // L):
                y1a = ay1[pl.ds(j * L, L)]
                x1a = ax1[pl.ds(j * L, L)]
                y2a = ay2[pl.ds(j * L, L)]
                x2a = ax2[pl.ds(j * L, L)]
                ara = aar[pl.ds(j * L, L)]
                ih = jnp.maximum(jnp.minimum(cy2, y2a) - jnp.maximum(cy1, y1a), 0.0)
                iw = jnp.maximum(jnp.minimum(cx2, x2a) - jnp.maximum(cx1, x1a), 0.0)
                inter = ih * iw
                union = car + ara - inter
                iou = inter / jnp.maximum(union, 1e-8)
                sup = jnp.where(iou > IOU_T, 1, sup)
            nsup = splat_max(sup)[0]
            accept = jnp.logical_and(valid, nsup == 0)

            @pl.when(valid)
            def _():
                l0n = jnp.where(iota == f0, negv, l0)
                sw[pl.ds(c * L, L)] = l0n
                l1n = jnp.where(iota == f1, splat_max(l0n), l1v)
                l1[pl.ds(k * L, L)] = l1n
                q2 = k // L * L
                r2 = k % L
                u2 = l2[pl.ds(q2, L)]
                l2[pl.ds(q2, L)] = jnp.where(iota == r2, splat_max(l1n), u2)

            @pl.when(accept)
            def _():
                qa = cnt // L * L
                mska = iota == (cnt % L)

                def put(ref, sv):
                    v = ref[pl.ds(qa, L)]
                    ref[pl.ds(qa, L)] = jnp.where(mska, sv, v)

                put(ay1, cy1)
                put(ax1, cx1)
                put(ay2, cy2)
                put(ax2, cx2)
                put(aar, car)
                put(outs, mv)
                c4 = cnt * 4
                qb = c4 // L * L
                lb = c4 % L
                vb = outb[pl.ds(qb, L)]
                vb = jnp.where(iota == lb, b0, vb)
                vb = jnp.where(iota == lb + 1, b1, vb)
                vb = jnp.where(iota == lb + 2, b2, vb)
                vb = jnp.where(iota == lb + 3, b3, vb)
                outb[pl.ds(qb, L)] = vb

            ncnt = cnt + jnp.where(accept, 1, 0).astype(jnp.int32)
            go = jnp.logical_and(valid, ncnt < MAXP)
            return (ncnt, jnp.where(go, 1, 0).astype(jnp.int32))

        def step(i, carry):
            return lax.cond(carry[1] > 0, body, lambda cr: cr, carry)

        state = (jnp.int32(0), jnp.int32(1))
        for rs in (192, 832, 3264, 16192):
            state = lax.cond(
                state[1] > 0,
                lambda s, n=rs: lax.fori_loop(0, n, step, s),
                lambda s: s,
                state,
            )
        pltpu.sync_copy(outs, sels_sh.at[pl.ds(lt * PP, PP)])
        pltpu.sync_copy(outb, selb_sh.at[pl.ds(lt * PP * 4, PP * 4)])

    def task_loop(r, bprev):
        t = sc * TPC + jnp.where(lid < 8, lid * 3 + r, 24 + (lid - 8) * 2 + r)
        act = jnp.logical_or(lid < 8, r < 2)
        b = t // NCLS
        bnew = jnp.where(act, b, bprev)

        @pl.when(act)
        def _():
            run_task(t, b, b != bprev)

        return bnew

    lax.fori_loop(0, 3, task_loop, jnp.int32(-1))

    plsc.subcore_barrier()

    @pl.when(lid < 2)
    def _():
        b = sc * 2 + lid
        pltpu.sync_copy(sels_sh.at[pl.ds(lid * M2, M2)], s2.at[pl.ds(0, M2)])
        pltpu.sync_copy(selb_sh.at[pl.ds(lid * M2 * 4, M2 * 4)], bx2)
        for j in range(M2 // L, M2PAD // L):
            s2[pl.ds(j * L, L)] = negv

        def build(q, carry):
            asm = negv
            for r in range(L):
                u = s2[pl.ds(q * 256 + r * L, L)]
                asm = jnp.where(iota == r, splat_max(u), asm)
            l1b[pl.ds(q * L, L)] = asm
            return carry

        lax.fori_loop(0, ML1 // L, build, jnp.int32(0))
        asm2 = negv
        for r in range(ML1 // L):
            u = l1b[pl.ds(r * L, L)]
            asm2 = jnp.where(iota == r, splat_max(u), asm2)
        l2b[pl.ds(0, L)] = asm2
        for j in range(304 // L):
            vouts[pl.ds(j * L, L)] = zerov
            voutc[pl.ds(j * L, L)] = zerov
        for j in range(1216 // L):
            voutb[pl.ds(j * L, L)] = zerov

        def mstep(i, cnt):
            w2 = l2b[pl.ds(0, L)]
            mv = splat_max(w2)
            k2 = splat_min(jnp.where(w2 == mv, iota, bigv))[0]
            l1v = l1b[pl.ds(k2 * L, L)]
            r1v = splat_min(jnp.where(l1v == mv, iota, bigv))
            r1 = r1v[0]
            k = k2 * L + r1
            l0 = s2[pl.ds(k * L, L)]
            f0 = splat_min(jnp.where(l0 == mv, iota, bigv))[0]
            idx = k * L + f0
            m = mv[0]
            valid = m > NEGV * 0.5

            @pl.when(valid)
            def _():
                l0n = jnp.where(iota == f0, negv, l0)
                s2[pl.ds(k * L, L)] = l0n
                l1n = jnp.where(iota == r1, splat_max(l0n), l1v)
                l1b[pl.ds(k2 * L, L)] = l1n
                l2b[pl.ds(0, L)] = jnp.where(iota == k2, splat_max(l1n), w2)
                cls = idx // PP
                clsv = jnp.full((L,), cls, jnp.int32).astype(jnp.float32)
                i4 = idx * 4
                qx = i4 // L * L
                lx = i4 % L
                vbx = bx2[pl.ds(qx, L)]
                vbx = jnp.minimum(jnp.maximum(vbx, 0.0), 1.0)
                b0 = lane(vbx, lx)
                b1 = lane(vbx, lx + 1)
                b2 = lane(vbx, lx + 2)
                b3 = lane(vbx, lx + 3)
                qo = i // L * L
                msko = iota == (i % L)
                vo = vouts[pl.ds(qo, L)]
                vouts[pl.ds(qo, L)] = jnp.where(msko, mv, vo)
                vc = voutc[pl.ds(qo, L)]
                voutc[pl.ds(qo, L)] = jnp.where(msko, clsv, vc)
                q4 = i * 4
                qb = q4 // L * L
                lb = q4 % L
                vb = voutb[pl.ds(qb, L)]
                vb = jnp.where(iota == lb, b0, vb)
                vb = jnp.where(iota == lb + 1, b1, vb)
                vb = jnp.where(iota == lb + 2, b2, vb)
                vb = jnp.where(iota == lb + 3, b3, vb)
                voutb[pl.ds(qb, L)] = vb

            return cnt + jnp.where(valid, 1, 0).astype(jnp.int32)

        cnt = lax.fori_loop(0, MAXT, mstep, jnp.int32(0))
        vcnt[pl.ds(0, L)] = jnp.where(iota == 0, jnp.full((L,), cnt, jnp.int32),
                                      jnp.zeros((L,), jnp.int32))
        pltpu.sync_copy(vouts, os_hbm.at[pl.ds(b * 304, 304)])
        pltpu.sync_copy(voutb, ob_hbm.at[pl.ds(b * 1216, 1216)])
        pltpu.sync_copy(voutc, ocl_hbm.at[pl.ds(b * 304, 304)])
        pltpu.sync_copy(vcnt, ocnt_hbm.at[pl.ds(b * L, L)])


def kernel(scores, boxes):
    scores_t = jnp.transpose(scores, (0, 2, 1)).reshape(-1)
    boxes_f = boxes[:, :, 0, :].reshape(-1)
    osf, obf, oclf, ocnt = _nms_fused(scores_t, boxes_f)
    out_scores = osf.reshape(NBATCH, 304)[:, :MAXT]
    out_boxes = obf.reshape(NBATCH, 304, 4)[:, :MAXT, :]
    out_classes = oclf.reshape(NBATCH, 304)[:, :MAXT]
    valid_counts = ocnt.reshape(NBATCH, 16)[:, 0]
    return (out_scores, out_boxes, out_classes, valid_counts)
